# baseline (speedup 1.0000x reference)
import functools
import math

import jax
import jax.numpy as jnp
from jax import lax
from jax.experimental import pallas as pl
from jax.experimental.pallas import tpu as pltpu
from jax.experimental.pallas import tpu_sc as plsc

_RAD = 15.0 * (math.pi / 180.0)

_NC, _NS, _L = 2, 16, 16
_NW = _NC * _NS
_CHUNK = 128


@functools.lru_cache(maxsize=None)
def _build(B, V, D):
    assert B % (_NW * _CHUNK) == 0, (B, _NW, _CHUNK)
    assert D >= _L
    b_per_w = B // _NW
    n_chunks = b_per_w // _CHUNK
    offs = list(range(0, D - _L + 1, _L))
    if offs[-1] != D - _L:
        offs.append(D - _L)
    mesh = plsc.VectorSubcoreMesh(core_axis_name="c", subcore_axis_name="s")

    @functools.partial(
        pl.kernel,
        mesh=mesh,
        out_type=jax.ShapeDtypeStruct((B, D), jnp.float32),
        scratch_types=[
            pltpu.VMEM((n_chunks, _CHUNK), jnp.int32),
            pltpu.VMEM((b_per_w, D), jnp.float32),
            pltpu.VMEM((_L,), jnp.float32),
            pltpu.SemaphoreType.DMA,
        ],
        compiler_params=pltpu.CompilerParams(use_tc_tiling_on_sc=False),
    )
    def gather_tanh(idx_hbm, table_hbm, res_hbm, out_hbm,
                    idx_v, rows_v, res_v, sem):
        wid = lax.axis_index("s") * _NC + lax.axis_index("c")
        base = wid * b_per_w
        pltpu.sync_copy(idx_hbm.at[pl.ds(wid * n_chunks, n_chunks)], idx_v)
        pltpu.sync_copy(res_hbm, res_v)
        for j in range(n_chunks):
            pltpu.async_copy(
                table_hbm.at[idx_v.at[j]],
                rows_v.at[pl.ds(j * _CHUNK, _CHUNK)],
                sem,
            ).wait()
        res = res_v[...]

        def row_body(r, carry):
            ws = [rows_v[r, pl.ds(o, _L)] for o in offs]
            for o, w in zip(offs, ws):
                t = jnp.exp(w * 2.0)
                rows_v[r, pl.ds(o, _L)] = (
                    _RAD - (2.0 * _RAD) / (t + 1.0)
                ) + res
            return carry

        lax.fori_loop(0, b_per_w, row_body, 0)
        pltpu.sync_copy(rows_v, out_hbm.at[pl.ds(base, b_per_w)])

    return gather_tanh


def kernel(sample_index, bones_rotations_weight, num_bones):
    B = sample_index.shape[0]
    _, D = bones_rotations_weight.shape
    nb = D // 3
    idx = sample_index.astype(jnp.int32).reshape(B // _CHUNK, _CHUNK)
    res = jnp.broadcast_to(
        jnp.asarray(num_bones, jnp.float32) - jnp.float32(nb), (_L,)
    )
    out = _build(B, bones_rotations_weight.shape[0], D)(
        idx, bones_rotations_weight, res
    )
    return out.reshape(B, nb, 3)


# speedup vs baseline: 1.5317x; 1.5317x over previous
import functools
import math

import jax
import jax.numpy as jnp
from jax import lax
from jax.experimental import pallas as pl
from jax.experimental.pallas import tpu as pltpu
from jax.experimental.pallas import tpu_sc as plsc

_RAD = 15.0 * (math.pi / 180.0)

_NC, _NS = 2, 16
_NW = _NC * _NS


@functools.lru_cache(maxsize=None)
def _build_gather(B, V, D):
    assert B % _NW == 0, (B, _NW)
    b_per_w = B // _NW
    mesh = plsc.VectorSubcoreMesh(core_axis_name="c", subcore_axis_name="s")

    @functools.partial(
        pl.kernel,
        mesh=mesh,
        out_type=jax.ShapeDtypeStruct((B, D), jnp.float32),
        scratch_types=[
            pltpu.VMEM((b_per_w,), jnp.int32),
            pltpu.SMEM((b_per_w,), jnp.int32),
            pltpu.SemaphoreType.DMA,
        ],
    )
    def gather_rows(idx_hbm, table_hbm, out_hbm, idx_v, idx_s, sem):
        wid = lax.axis_index("s") * _NC + lax.axis_index("c")
        base = wid * b_per_w
        pltpu.sync_copy(idx_hbm.at[pl.ds(base, b_per_w)], idx_v)
        for g in range(b_per_w // 16):
            vec = idx_v[pl.ds(g * 16, 16)]
            for k in range(16):
                idx_s[g * 16 + k] = vec[k]

        def enqueue(i, carry):
            row = idx_s[i]
            pltpu.async_copy(
                table_hbm.at[pl.ds(row, 1)],
                out_hbm.at[pl.ds(base + i, 1)],
                sem,
            )
            return carry

        lax.fori_loop(0, b_per_w, enqueue, 0)

        def drain(i, carry):
            pltpu.make_async_copy(
                table_hbm.at[pl.ds(0, 1)],
                out_hbm.at[pl.ds(base, 1)],
                sem,
            ).wait()
            return carry

        lax.fori_loop(0, b_per_w, drain, 0)

    return gather_rows


@functools.lru_cache(maxsize=None)
def _build_tanh(B, D, rows_per_block):
    grid = B // rows_per_block

    def tanh_body(res_ref, x_ref, o_ref):
        o_ref[...] = jnp.tanh(x_ref[...]) * _RAD + res_ref[0]

    return pl.pallas_call(
        tanh_body,
        grid=(grid,),
        in_specs=[
            pl.BlockSpec(memory_space=pltpu.SMEM),
            pl.BlockSpec((rows_per_block, D), lambda i: (i, 0)),
        ],
        out_specs=pl.BlockSpec((rows_per_block, D), lambda i: (i, 0)),
        out_shape=jax.ShapeDtypeStruct((B, D), jnp.float32),
    )


def kernel(sample_index, bones_rotations_weight, num_bones):
    B = sample_index.shape[0]
    V, D = bones_rotations_weight.shape
    nb = D // 3
    idx = sample_index.astype(jnp.int32)
    res = jnp.reshape(jnp.asarray(num_bones, jnp.float32) - jnp.float32(nb), (1,))
    gathered = _build_gather(B, V, D)(idx, bones_rotations_weight)
    out = _build_tanh(B, D, 2048)(res, gathered)
    return out.reshape(B, nb, 3)
